# baseline (speedup 1.0000x reference)
import functools

import jax
import jax.numpy as jnp
from jax import lax
from jax.experimental import pallas as pl
from jax.experimental.pallas import tpu as pltpu
from jax.experimental.pallas import tpu_sc as plsc

_VOCAB = 1000000
_RANK = 32
_DIM = 64
_BATCH = 16384
_HIST = 50
_NQ = _HIST // 4
_NROWS = _BATCH * (_NQ * 4 + 4)

_NC = 2
_NS = 16
_NW = _NC * _NS
_CHUNK = 128
_NCH = _NROWS // (_NW * _CHUNK)


def _sc_gather(idx_hbm, table_hbm, out_hbm, idx_v, rows_v, sem):
    wid = lax.axis_index("s") * _NC + lax.axis_index("c")
    pltpu.sync_copy(idx_hbm.at[wid], idx_v)

    def body(j, carry):
        pltpu.async_copy(table_hbm.at[idx_v.at[j]], rows_v, sem).wait()
        pltpu.sync_copy(rows_v, out_hbm.at[wid * _NCH + j])
        return carry

    lax.fori_loop(0, _NCH, body, 0, unroll=False)


_sc_gather_call = functools.partial(
    pl.kernel,
    out_type=jax.ShapeDtypeStruct((_NW * _NCH, _CHUNK, _RANK), jnp.float32),
    mesh=plsc.VectorSubcoreMesh(core_axis_name="c", subcore_axis_name="s"),
    scratch_types=[
        pltpu.VMEM((_NCH, _CHUNK), jnp.int32),
        pltpu.VMEM((_CHUNK, _RANK), jnp.float32),
        pltpu.SemaphoreType.DMA,
    ],
    compiler_params=pltpu.CompilerParams(use_tc_tiling_on_sc=False),
)(_sc_gather)


_BM = 1024
_OUTW = _HIST * _DIM


def _tc_matmul_body(emb_ref, bp_ref, bpt_ref, out_ref):
    for q in range(_NQ):
        out_ref[:, q * 256:(q + 1) * 256] = lax.dot_general(
            emb_ref[:, q, :], bp_ref[...],
            (((1,), (0,)), ((), ())),
            preferred_element_type=jnp.float32,
        )
    out_ref[:, _NQ * 256:_OUTW] = lax.dot_general(
        emb_ref[:, _NQ, :], bpt_ref[...],
        (((1,), (0,)), ((), ())),
        preferred_element_type=jnp.float32,
    )


def _tc_matmul(emb_p, Bp, Bpt):
    return pl.pallas_call(
        _tc_matmul_body,
        grid=(_BATCH // _BM,),
        in_specs=[
            pl.BlockSpec((_BM, _NQ + 1, 128), lambda i: (i, 0, 0)),
            pl.BlockSpec((128, 256), lambda i: (0, 0)),
            pl.BlockSpec((128, 128), lambda i: (0, 0)),
        ],
        out_specs=pl.BlockSpec((_BM, _OUTW), lambda i: (i, 0)),
        out_shape=jax.ShapeDtypeStruct((_BATCH, _OUTW), jnp.float32),
        compiler_params=pltpu.CompilerParams(
            dimension_semantics=("arbitrary",),
        ),
    )(emb_p, Bp, Bpt)


def kernel(token_ids, A, B):
    tok = token_ids.astype(jnp.int32)
    idx = jnp.concatenate([tok, tok[:, _HIST - 2:]], axis=1)
    idx = idx.reshape(_NW, _NCH, _CHUNK)
    emb = _sc_gather_call(idx, A)
    emb_p = emb.reshape(_BATCH, _NQ + 1, 128)
    eye4 = jnp.eye(4, dtype=B.dtype)
    Bp = (eye4[:, None, :, None] * B[None, :, None, :]).reshape(128, 256)
    eye2 = jnp.eye(2, dtype=B.dtype)
    Bp2 = (eye2[:, None, :, None] * B[None, :, None, :]).reshape(64, 128)
    Bpt = jnp.concatenate([Bp2, jnp.zeros((64, 128), B.dtype)], axis=0)
    out = _tc_matmul(emb_p, Bp, Bpt)
    return out.reshape(_BATCH, _HIST, _DIM)


# speedup vs baseline: 1.2163x; 1.2163x over previous
import functools

import jax
import jax.numpy as jnp
from jax import lax
from jax.experimental import pallas as pl
from jax.experimental.pallas import tpu as pltpu
from jax.experimental.pallas import tpu_sc as plsc

_VOCAB = 1000000
_RANK = 32
_DIM = 64
_BATCH = 16384
_HIST = 50
_TOTAL = _BATCH * _HIST
_NPACK = 128 // _RANK
_GP = _BATCH // _NPACK

_NC = 2
_NS = 16
_NW = _NC * _NS
_CHUNK = 128
_G = 2
_HG = _HIST // _G
_NCHG = _HG * _BATCH // (_NW * _CHUNK)


def _sc_gather(idx_hbm, table_hbm, out_hbm, idx_v, rows_v, sem):
    wid = lax.axis_index("s") * _NC + lax.axis_index("c")
    pltpu.sync_copy(idx_hbm.at[wid], idx_v)

    def body(j, carry):
        pltpu.async_copy(table_hbm.at[idx_v.at[j]], rows_v, sem).wait()
        pltpu.sync_copy(rows_v, out_hbm.at[wid * _NCHG + j])
        return carry

    lax.fori_loop(0, _NCHG, body, 0, unroll=False)


_sc_gather_call = functools.partial(
    pl.kernel,
    out_type=jax.ShapeDtypeStruct((_NW * _NCHG, _CHUNK, _RANK), jnp.float32),
    mesh=plsc.VectorSubcoreMesh(core_axis_name="c", subcore_axis_name="s"),
    scratch_types=[
        pltpu.VMEM((_NCHG, _CHUNK), jnp.int32),
        pltpu.VMEM((_CHUNK, _RANK), jnp.float32),
        pltpu.SemaphoreType.DMA,
    ],
    compiler_params=pltpu.CompilerParams(use_tc_tiling_on_sc=False),
)(_sc_gather)


def _tc_matmul_body(emb_ref, bp_ref, out_ref):
    t = lax.dot_general(
        bp_ref[...], emb_ref[0],
        (((0,), (1,)), ((), ())),
        preferred_element_type=jnp.float32,
    )
    for a in range(_NPACK):
        out_ref[0, :, a * _GP:(a + 1) * _GP] = t[a * _DIM:(a + 1) * _DIM, :]


def _tc_matmul(emb_p, Bp):
    return pl.pallas_call(
        _tc_matmul_body,
        grid=(_HG,),
        in_specs=[
            pl.BlockSpec((1, _GP, 128), lambda h: (h, 0, 0)),
            pl.BlockSpec((128, _NPACK * _DIM), lambda h: (0, 0)),
        ],
        out_specs=pl.BlockSpec((1, _DIM, _BATCH), lambda h: (h, 0, 0)),
        out_shape=jax.ShapeDtypeStruct((_HG, _DIM, _BATCH), jnp.float32),
        compiler_params=pltpu.CompilerParams(
            dimension_semantics=("arbitrary",),
        ),
    )(emb_p, Bp)


def kernel(token_ids, A, B):
    tokT = token_ids.T.reshape(_HIST, _NPACK, 128, 32)
    idx = tokT.transpose(0, 2, 3, 1).reshape(_G, _NW, _NCHG, _CHUNK)
    idx = idx.astype(jnp.int32)
    eye = jnp.eye(_NPACK, dtype=B.dtype)
    Bp = (eye[:, None, :, None] * B[None, :, None, :]).reshape(
        _NPACK * _RANK, _NPACK * _DIM
    )
    outs = []
    for g in range(_G):
        emb = _sc_gather_call(idx[g], A)
        emb_p = emb.reshape(_HG, _GP, 128)
        outs.append(_tc_matmul(emb_p, Bp))
    out_t = jnp.concatenate(outs, axis=0)
    return out_t.transpose(2, 0, 1)


# speedup vs baseline: 1.3810x; 1.1354x over previous
import functools

import jax
import jax.numpy as jnp
from jax import lax
from jax.experimental import pallas as pl
from jax.experimental.pallas import tpu as pltpu
from jax.experimental.pallas import tpu_sc as plsc

_VOCAB = 1000000
_RANK = 32
_DIM = 64
_BATCH = 16384
_HIST = 50
_TOTAL = _BATCH * _HIST
_NPACK = 128 // _RANK
_GP = _BATCH // _NPACK

_NC = 2
_NS = 16
_NW = _NC * _NS
_CHUNK = 128
_G = 1
_HG = _HIST // _G
_NCHG = _HG * _BATCH // (_NW * _CHUNK)


def _sc_gather(idx_hbm, table_hbm, out_hbm, idx_v, rows_v, sem):
    wid = lax.axis_index("s") * _NC + lax.axis_index("c")
    pltpu.sync_copy(idx_hbm.at[wid], idx_v)

    def body(j, carry):
        pltpu.async_copy(table_hbm.at[idx_v.at[j]], rows_v, sem).wait()
        pltpu.sync_copy(rows_v, out_hbm.at[wid * _NCHG + j])
        return carry

    lax.fori_loop(0, _NCHG, body, 0, unroll=False)


_sc_gather_call = functools.partial(
    pl.kernel,
    out_type=jax.ShapeDtypeStruct((_NW * _NCHG, _CHUNK, _RANK), jnp.float32),
    mesh=plsc.VectorSubcoreMesh(core_axis_name="c", subcore_axis_name="s"),
    scratch_types=[
        pltpu.VMEM((_NCHG, _CHUNK), jnp.int32),
        pltpu.VMEM((_CHUNK, _RANK), jnp.float32),
        pltpu.SemaphoreType.DMA,
    ],
    compiler_params=pltpu.CompilerParams(use_tc_tiling_on_sc=False),
)(_sc_gather)


def _tc_matmul_body(emb_ref, bp_ref, out_ref):
    t = lax.dot_general(
        bp_ref[...], emb_ref[0],
        (((0,), (1,)), ((), ())),
        preferred_element_type=jnp.float32,
    )
    for a in range(_NPACK):
        out_ref[0, :, a * _GP:(a + 1) * _GP] = t[a * _DIM:(a + 1) * _DIM, :]


def _tc_matmul(emb_p, Bp):
    return pl.pallas_call(
        _tc_matmul_body,
        grid=(_HG,),
        in_specs=[
            pl.BlockSpec((1, _GP, 128), lambda h: (h, 0, 0)),
            pl.BlockSpec((128, _NPACK * _DIM), lambda h: (0, 0)),
        ],
        out_specs=pl.BlockSpec((1, _DIM, _BATCH), lambda h: (h, 0, 0)),
        out_shape=jax.ShapeDtypeStruct((_HG, _DIM, _BATCH), jnp.float32),
        compiler_params=pltpu.CompilerParams(
            dimension_semantics=("arbitrary",),
        ),
    )(emb_p, Bp)


def kernel(token_ids, A, B):
    tokT = token_ids.T.reshape(_HIST, _NPACK, 128, 32)
    idx = tokT.transpose(0, 2, 3, 1).reshape(_G, _NW, _NCHG, _CHUNK)
    idx = idx.astype(jnp.int32)
    eye = jnp.eye(_NPACK, dtype=B.dtype)
    Bp = (eye[:, None, :, None] * B[None, :, None, :]).reshape(
        _NPACK * _RANK, _NPACK * _DIM
    )
    outs = []
    for g in range(_G):
        emb = _sc_gather_call(idx[g], A)
        emb_p = emb.reshape(_HG, _GP, 128)
        outs.append(_tc_matmul(emb_p, Bp))
    out_t = jnp.concatenate(outs, axis=0)
    return out_t.transpose(2, 0, 1)
